# baseline (speedup 1.0000x reference)
import functools

import jax
import jax.numpy as jnp
from jax.experimental import pallas as pl
from jax.experimental.pallas import tpu as pltpu

_KNN = 10
_SIGMA = 1.0
_ALPHA = 0.99
_N = 4096
_D = 128
_BLK = 512
_NB = _N // _BLK
_BIGI = 2**30
_PREC = jax.lax.Precision.HIGHEST


def _dist_tile(qblk, kblk, k2row):
    q2 = jnp.sum(qblk * qblk, axis=1, keepdims=True)
    mm = jax.lax.dot_general(
        qblk, kblk, (((1,), (1,)), ((), ())),
        preferred_element_type=jnp.float32, precision=_PREC)
    return q2 + k2row - 2.0 * mm


def _norms_kernel(k_ref, k2_ref):
    k = k_ref[...]
    k2_ref[...] = jnp.sum(k * k, axis=1, keepdims=True)


def _stats_kernel(q_ref, k_ref, k2row_ref,
                  rs_ref, dg_ref, cs_ref, w_ref,
                  work_ref):
    bi = pl.program_id(0)
    dist = _dist_tile(q_ref[...], k_ref[...], k2row_ref[...])
    w_ref[...] = dist
    work_ref[...] = dist

    jcol = jax.lax.broadcasted_iota(jnp.int32, (_BLK, _N), 1)
    d1 = j1 = d11 = j11 = None
    for it in range(_KNN + 1):
        wk = work_ref[...]
        m = jnp.min(wk, axis=1, keepdims=True)
        hit = wk == m
        jm = jnp.min(jnp.where(hit, jcol, _BIGI), axis=1, keepdims=True)
        if it == 0:
            d1, j1 = m, jm
        if it == _KNN:
            d11, j11 = m, jm
        if it < _KNN:
            work_ref[...] = jnp.where(jcol == jm, jnp.inf, wk)

    dist = w_ref[...]
    sel = ((dist < d11) | ((dist == d11) & (jcol <= j11))) & \
          ((dist > d1) | ((dist == d1) & (jcol > j1)))
    w = jnp.where(sel, jnp.exp(-dist / (_SIGMA ** 2.0)), 0.0)
    w_ref[...] = w
    rs_ref[...] = jnp.sum(w, axis=1, keepdims=True)
    irow = jax.lax.broadcasted_iota(jnp.int32, (_BLK, _N), 0) + bi * _BLK
    dg_ref[...] = jnp.sum(jnp.where(jcol == irow, w, 0.0),
                          axis=1, keepdims=True)
    cpart = jnp.sum(w, axis=0, keepdims=True)

    @pl.when(bi == 0)
    def _init():
        cs_ref[...] = cpart

    @pl.when(bi > 0)
    def _acc():
        cs_ref[...] = cs_ref[...] + cpart


def _dn(rs, cs, dg):
    s = rs + cs - 2.0 * dg
    s = jnp.where(s == 0.0, 1.0, s)
    return 1.0 / jnp.sqrt(s)


_BR = 512
_BC = 1024
_NBR = _N // _BR
_NBC = _N // _BC


def _assemble_kernel(w1_ref, w2_ref,
                     rsi_ref, csi_ref, dgi_ref,
                     rsj_ref, csj_ref, dgj_ref,
                     a_ref):
    bi = pl.program_id(0)
    bj = pl.program_id(1)
    w2t = jnp.transpose(w2_ref[...])
    dni = _dn(rsi_ref[...], csi_ref[...], dgi_ref[...])
    dnj = _dn(rsj_ref[...].reshape(1, _BC),
              csj_ref[...].reshape(1, _BC),
              dgj_ref[...].reshape(1, _BC))
    offd = -_ALPHA * (dni * dnj) * (w1_ref[...] + w2t)
    irow = jax.lax.broadcasted_iota(jnp.int32, (_BR, _BC), 0) + bi * _BR
    jcol = jax.lax.broadcasted_iota(jnp.int32, (_BR, _BC), 1) + bj * _BC
    a_ref[...] = jnp.where(irow == jcol, 1.0, offd)


@jax.jit
def kernel(queries, keys):
    f32 = jnp.float32
    k2c = pl.pallas_call(
        _norms_kernel,
        out_shape=jax.ShapeDtypeStruct((_N, 1), f32),
    )(keys)
    k2row = k2c.reshape(1, _N)

    col = pl.BlockSpec((_BLK, 1), lambda bi: (bi, 0))
    stats = pl.pallas_call(
        _stats_kernel,
        grid=(_NB,),
        in_specs=[
            pl.BlockSpec((_BLK, _D), lambda bi: (bi, 0)),
            pl.BlockSpec((_N, _D), lambda bi: (0, 0)),
            pl.BlockSpec((1, _N), lambda bi: (0, 0)),
        ],
        out_specs=[col, col,
                   pl.BlockSpec((1, _N), lambda bi: (0, 0)),
                   pl.BlockSpec((_BLK, _N), lambda bi: (bi, 0))],
        out_shape=[jax.ShapeDtypeStruct((_N, 1), f32),
                   jax.ShapeDtypeStruct((_N, 1), f32),
                   jax.ShapeDtypeStruct((1, _N), f32),
                   jax.ShapeDtypeStruct((_N, _N), f32)],
        scratch_shapes=[pltpu.VMEM((_BLK, _N), f32)],
        compiler_params=pltpu.CompilerParams(
            dimension_semantics=("arbitrary",)),
    )(queries, keys, k2row)
    rsc, dgc, csrow, w0 = stats
    csc = csrow.reshape(_N, 1)
    rsr = rsc.reshape(_NBC, 1, _BC)
    csr = csc.reshape(_NBC, 1, _BC)
    dgr = dgc.reshape(_NBC, 1, _BC)

    col_i = pl.BlockSpec((_BR, 1), lambda bi, bj: (bi, 0))
    row_j = pl.BlockSpec((1, 1, _BC), lambda bi, bj: (bj, 0, 0))

    a = pl.pallas_call(
        _assemble_kernel,
        grid=(_NBR, _NBC),
        in_specs=[pl.BlockSpec((_BR, _BC), lambda bi, bj: (bi, bj)),
                  pl.BlockSpec((_BC, _BR), lambda bi, bj: (bj, bi)),
                  col_i, col_i, col_i,
                  row_j, row_j, row_j],
        out_specs=pl.BlockSpec((_BR, _BC), lambda bi, bj: (bi, bj)),
        out_shape=jax.ShapeDtypeStruct((_N, _N), f32),
        compiler_params=pltpu.CompilerParams(
            dimension_semantics=("arbitrary", "arbitrary")),
    )(w0, w0,
      rsc, csc, dgc,
      rsr, csr, dgr)
    return a


# speedup vs baseline: 1.3540x; 1.3540x over previous
import functools

import jax
import jax.numpy as jnp
from jax.experimental import pallas as pl
from jax.experimental.pallas import tpu as pltpu

_KNN = 10
_SIGMA = 1.0
_ALPHA = 0.99
_N = 4096
_D = 128
_BLK = 256
_NB = _N // _BLK
_BIGI = 2**30
_PREC = jax.lax.Precision.HIGHEST


def _dist_tile(qblk, kblk, k2row):
    q2 = jnp.sum(qblk * qblk, axis=1, keepdims=True)
    mm = jax.lax.dot_general(
        qblk, kblk, (((1,), (1,)), ((), ())),
        preferred_element_type=jnp.float32, precision=_PREC)
    return q2 + k2row - 2.0 * mm


def _norms_kernel(k_ref, k2_ref):
    k = k_ref[...]
    k2_ref[...] = jnp.sum(k * k, axis=1, keepdims=True)


def _stats_kernel(q_ref, k_ref, k2row_ref,
                  rs_ref, dg_ref, cs_ref, w_ref,
                  work_ref):
    bi = pl.program_id(0)
    dist = _dist_tile(q_ref[...], k_ref[...], k2row_ref[...])
    w_ref[...] = dist
    work_ref[...] = dist

    jcol = jax.lax.broadcasted_iota(jnp.int32, (_BLK, _N), 1).astype(jnp.float32)
    d1 = j1 = d11 = j11 = None
    for it in range(_KNN + 1):
        wk = work_ref[...]
        m = jnp.min(wk, axis=1, keepdims=True)
        hit = wk == m
        jm = jnp.min(jnp.where(hit, jcol, jnp.inf), axis=1, keepdims=True)
        if it == 0:
            d1, j1 = m, jm
        if it == _KNN:
            d11, j11 = m, jm
        if it < _KNN:
            work_ref[...] = jnp.where(jcol == jm, jnp.inf, wk)

    dist = w_ref[...]
    sel = ((dist < d11) | ((dist == d11) & (jcol <= j11))) & \
          ((dist > d1) | ((dist == d1) & (jcol > j1)))
    w = jnp.where(sel, jnp.exp(-dist / (_SIGMA ** 2.0)), 0.0)
    w_ref[...] = w
    rs_ref[...] = jnp.sum(w, axis=1, keepdims=True)
    irow = (jax.lax.broadcasted_iota(jnp.int32, (_BLK, _N), 0) + bi * _BLK).astype(jnp.float32)
    dg_ref[...] = jnp.sum(jnp.where(jcol == irow, w, 0.0),
                          axis=1, keepdims=True)
    cpart = jnp.sum(w, axis=0, keepdims=True)

    @pl.when(bi == 0)
    def _init():
        cs_ref[...] = cpart

    @pl.when(bi > 0)
    def _acc():
        cs_ref[...] = cs_ref[...] + cpart


def _dn(rs, cs, dg):
    s = rs + cs - 2.0 * dg
    s = jnp.where(s == 0.0, 1.0, s)
    return 1.0 / jnp.sqrt(s)


_BR = 512
_BC = 1024
_NBR = _N // _BR
_NBC = _N // _BC


def _assemble_kernel(w1_ref, w2_ref,
                     rsi_ref, csi_ref, dgi_ref,
                     rsj_ref, csj_ref, dgj_ref,
                     a_ref):
    bi = pl.program_id(0)
    bj = pl.program_id(1)
    w2t = jnp.transpose(w2_ref[...])
    dni = _dn(rsi_ref[...], csi_ref[...], dgi_ref[...])
    dnj = _dn(rsj_ref[...].reshape(1, _BC),
              csj_ref[...].reshape(1, _BC),
              dgj_ref[...].reshape(1, _BC))
    offd = -_ALPHA * (dni * dnj) * (w1_ref[...] + w2t)
    irow = jax.lax.broadcasted_iota(jnp.int32, (_BR, _BC), 0) + bi * _BR
    jcol = jax.lax.broadcasted_iota(jnp.int32, (_BR, _BC), 1) + bj * _BC
    a_ref[...] = jnp.where(irow == jcol, 1.0, offd)


@jax.jit
def kernel(queries, keys):
    f32 = jnp.float32
    k2c = pl.pallas_call(
        _norms_kernel,
        out_shape=jax.ShapeDtypeStruct((_N, 1), f32),
    )(keys)
    k2row = k2c.reshape(1, _N)

    col = pl.BlockSpec((_BLK, 1), lambda bi: (bi, 0))
    stats = pl.pallas_call(
        _stats_kernel,
        grid=(_NB,),
        in_specs=[
            pl.BlockSpec((_BLK, _D), lambda bi: (bi, 0)),
            pl.BlockSpec((_N, _D), lambda bi: (0, 0)),
            pl.BlockSpec((1, _N), lambda bi: (0, 0)),
        ],
        out_specs=[col, col,
                   pl.BlockSpec((1, _N), lambda bi: (0, 0)),
                   pl.BlockSpec((_BLK, _N), lambda bi: (bi, 0))],
        out_shape=[jax.ShapeDtypeStruct((_N, 1), f32),
                   jax.ShapeDtypeStruct((_N, 1), f32),
                   jax.ShapeDtypeStruct((1, _N), f32),
                   jax.ShapeDtypeStruct((_N, _N), f32)],
        scratch_shapes=[pltpu.VMEM((_BLK, _N), f32)],
        compiler_params=pltpu.CompilerParams(
            dimension_semantics=("arbitrary",)),
    )(queries, keys, k2row)
    rsc, dgc, csrow, w0 = stats
    csc = csrow.reshape(_N, 1)
    rsr = rsc.reshape(_NBC, 1, _BC)
    csr = csc.reshape(_NBC, 1, _BC)
    dgr = dgc.reshape(_NBC, 1, _BC)

    col_i = pl.BlockSpec((_BR, 1), lambda bi, bj: (bi, 0))
    row_j = pl.BlockSpec((1, 1, _BC), lambda bi, bj: (bj, 0, 0))

    a = pl.pallas_call(
        _assemble_kernel,
        grid=(_NBR, _NBC),
        in_specs=[pl.BlockSpec((_BR, _BC), lambda bi, bj: (bi, bj)),
                  pl.BlockSpec((_BC, _BR), lambda bi, bj: (bj, bi)),
                  col_i, col_i, col_i,
                  row_j, row_j, row_j],
        out_specs=pl.BlockSpec((_BR, _BC), lambda bi, bj: (bi, bj)),
        out_shape=jax.ShapeDtypeStruct((_N, _N), f32),
        compiler_params=pltpu.CompilerParams(
            dimension_semantics=("arbitrary", "arbitrary")),
    )(w0, w0,
      rsc, csc, dgc,
      rsr, csr, dgr)
    return a


# speedup vs baseline: 1.3999x; 1.0339x over previous
import functools

import jax
import jax.numpy as jnp
from jax.experimental import pallas as pl
from jax.experimental.pallas import tpu as pltpu

_KNN = 10
_SIGMA = 1.0
_ALPHA = 0.99
_N = 4096
_D = 128
_BLK = 256
_NB = _N // _BLK
_BIGI = 2**30
_PREC = jax.lax.Precision.HIGHEST


def _dist_tile(qblk, kblk, k2row):
    q2 = jnp.sum(qblk * qblk, axis=1, keepdims=True)
    mm = jax.lax.dot_general(
        qblk, kblk, (((1,), (1,)), ((), ())),
        preferred_element_type=jnp.float32, precision=_PREC)
    return q2 + k2row - 2.0 * mm


def _norms_kernel(k_ref, k2_ref):
    k = k_ref[...]
    k2_ref[...] = jnp.sum(k * k, axis=1, keepdims=True)


def _stats_kernel(q_ref, k_ref, k2row_ref,
                  rs_ref, dg_ref, cs_ref, w_ref,
                  work_ref):
    bi = pl.program_id(0)
    dist = _dist_tile(q_ref[...], k_ref[...], k2row_ref[...])
    w_ref[...] = dist
    work_ref[...] = dist

    jcol = jax.lax.broadcasted_iota(jnp.int32, (_BLK, _N), 1).astype(jnp.float32)
    d1 = j1 = d11 = j11 = None
    for it in range(_KNN + 1):
        wk = work_ref[...]
        m = jnp.min(wk, axis=1, keepdims=True)
        hit = wk == m
        jm = jnp.min(jnp.where(hit, jcol, jnp.inf), axis=1, keepdims=True)
        if it == 0:
            d1, j1 = m, jm
        if it == _KNN:
            d11, j11 = m, jm
        if it < _KNN:
            work_ref[...] = jnp.where(jcol == jm, jnp.inf, wk)

    wkf = work_ref[...]
    dist = w_ref[...]
    sel = (jnp.isinf(wkf) & (jcol != j1)) | ((wkf == d11) & (jcol == j11))
    w = jnp.where(sel, jnp.exp(-dist / (_SIGMA ** 2.0)), 0.0)
    w_ref[...] = w
    rs_ref[...] = jnp.sum(w, axis=1, keepdims=True)
    irow = (jax.lax.broadcasted_iota(jnp.int32, (_BLK, _N), 0) + bi * _BLK).astype(jnp.float32)
    dg_ref[...] = jnp.sum(jnp.where(jcol == irow, w, 0.0),
                          axis=1, keepdims=True)
    cpart = jnp.sum(w, axis=0, keepdims=True)

    @pl.when(bi == 0)
    def _init():
        cs_ref[...] = cpart

    @pl.when(bi > 0)
    def _acc():
        cs_ref[...] = cs_ref[...] + cpart


def _dn(rs, cs, dg):
    s = rs + cs - 2.0 * dg
    s = jnp.where(s == 0.0, 1.0, s)
    return 1.0 / jnp.sqrt(s)


_BR = 512
_BC = 1024
_NBR = _N // _BR
_NBC = _N // _BC


def _assemble_kernel(w1_ref, w2_ref,
                     rsi_ref, csi_ref, dgi_ref,
                     rsj_ref, csj_ref, dgj_ref,
                     a_ref):
    bi = pl.program_id(0)
    bj = pl.program_id(1)
    w2t = jnp.transpose(w2_ref[...])
    dni = _dn(rsi_ref[...], csi_ref[...], dgi_ref[...])
    dnj = _dn(rsj_ref[...].reshape(1, _BC),
              csj_ref[...].reshape(1, _BC),
              dgj_ref[...].reshape(1, _BC))
    offd = -_ALPHA * (dni * dnj) * (w1_ref[...] + w2t)
    irow = jax.lax.broadcasted_iota(jnp.int32, (_BR, _BC), 0) + bi * _BR
    jcol = jax.lax.broadcasted_iota(jnp.int32, (_BR, _BC), 1) + bj * _BC
    a_ref[...] = jnp.where(irow == jcol, 1.0, offd)


@jax.jit
def kernel(queries, keys):
    f32 = jnp.float32
    k2c = pl.pallas_call(
        _norms_kernel,
        out_shape=jax.ShapeDtypeStruct((_N, 1), f32),
    )(keys)
    k2row = k2c.reshape(1, _N)

    col = pl.BlockSpec((_BLK, 1), lambda bi: (bi, 0))
    stats = pl.pallas_call(
        _stats_kernel,
        grid=(_NB,),
        in_specs=[
            pl.BlockSpec((_BLK, _D), lambda bi: (bi, 0)),
            pl.BlockSpec((_N, _D), lambda bi: (0, 0)),
            pl.BlockSpec((1, _N), lambda bi: (0, 0)),
        ],
        out_specs=[col, col,
                   pl.BlockSpec((1, _N), lambda bi: (0, 0)),
                   pl.BlockSpec((_BLK, _N), lambda bi: (bi, 0))],
        out_shape=[jax.ShapeDtypeStruct((_N, 1), f32),
                   jax.ShapeDtypeStruct((_N, 1), f32),
                   jax.ShapeDtypeStruct((1, _N), f32),
                   jax.ShapeDtypeStruct((_N, _N), f32)],
        scratch_shapes=[pltpu.VMEM((_BLK, _N), f32)],
        compiler_params=pltpu.CompilerParams(
            dimension_semantics=("arbitrary",)),
    )(queries, keys, k2row)
    rsc, dgc, csrow, w0 = stats
    csc = csrow.reshape(_N, 1)
    rsr = rsc.reshape(_NBC, 1, _BC)
    csr = csc.reshape(_NBC, 1, _BC)
    dgr = dgc.reshape(_NBC, 1, _BC)

    col_i = pl.BlockSpec((_BR, 1), lambda bi, bj: (bi, 0))
    row_j = pl.BlockSpec((1, 1, _BC), lambda bi, bj: (bj, 0, 0))

    a = pl.pallas_call(
        _assemble_kernel,
        grid=(_NBR, _NBC),
        in_specs=[pl.BlockSpec((_BR, _BC), lambda bi, bj: (bi, bj)),
                  pl.BlockSpec((_BC, _BR), lambda bi, bj: (bj, bi)),
                  col_i, col_i, col_i,
                  row_j, row_j, row_j],
        out_specs=pl.BlockSpec((_BR, _BC), lambda bi, bj: (bi, bj)),
        out_shape=jax.ShapeDtypeStruct((_N, _N), f32),
        compiler_params=pltpu.CompilerParams(
            dimension_semantics=("arbitrary", "arbitrary")),
    )(w0, w0,
      rsc, csc, dgc,
      rsr, csr, dgr)
    return a


# speedup vs baseline: 1.4711x; 1.0509x over previous
import functools

import jax
import jax.numpy as jnp
from jax.experimental import pallas as pl
from jax.experimental.pallas import tpu as pltpu

_KNN = 10
_SIGMA = 1.0
_ALPHA = 0.99
_N = 4096
_D = 128
_BLK = 256
_NB = _N // _BLK
_BIGI = 2**30
_PREC = jax.lax.Precision.HIGHEST


def _dist_tile(qblk, kblk, k2row):
    q2 = jnp.sum(qblk * qblk, axis=1, keepdims=True)
    mm = jax.lax.dot_general(
        qblk, kblk, (((1,), (1,)), ((), ())),
        preferred_element_type=jnp.float32, precision=_PREC)
    return q2 + k2row - 2.0 * mm


def _norms_kernel(k_ref, k2_ref):
    k = k_ref[...]
    k2_ref[...] = jnp.sum(k * k, axis=1, keepdims=True)


def _stats_kernel(q_ref, k_ref, k2row_ref,
                  rs_ref, dg_ref, cs_ref, w_ref,
                  dist_ref, work_ref):
    bi = pl.program_id(0)
    dist = _dist_tile(q_ref[...], k_ref[...], k2row_ref[...])
    dist_ref[...] = dist
    work_ref[...] = dist

    jcol = jax.lax.broadcasted_iota(jnp.int32, (_BLK, _N), 1).astype(jnp.float32)
    d1 = j1 = d11 = j11 = None
    for it in range(_KNN + 1):
        wk = work_ref[...]
        m = jnp.min(wk, axis=1, keepdims=True)
        hit = wk == m
        jm = jnp.min(jnp.where(hit, jcol, jnp.inf), axis=1, keepdims=True)
        if it == 0:
            d1, j1 = m, jm
        if it == _KNN:
            d11, j11 = m, jm
        if it < _KNN:
            work_ref[...] = jnp.where(jcol == jm, jnp.inf, wk)

    wkf = work_ref[...]
    dist = dist_ref[...]
    sel = (jnp.isinf(wkf) & (jcol != j1)) | ((wkf == d11) & (jcol == j11))
    w = jnp.where(sel, jnp.exp(-dist / (_SIGMA ** 2.0)), 0.0)
    w_ref[...] = w.astype(jnp.bfloat16)
    rs_ref[...] = jnp.sum(w, axis=1, keepdims=True)
    irow = (jax.lax.broadcasted_iota(jnp.int32, (_BLK, _N), 0) + bi * _BLK).astype(jnp.float32)
    dg_ref[...] = jnp.sum(jnp.where(jcol == irow, w, 0.0),
                          axis=1, keepdims=True)
    cpart = jnp.sum(w, axis=0, keepdims=True)

    @pl.when(bi == 0)
    def _init():
        cs_ref[...] = cpart

    @pl.when(bi > 0)
    def _acc():
        cs_ref[...] = cs_ref[...] + cpart


def _dn(rs, cs, dg):
    s = rs + cs - 2.0 * dg
    s = jnp.where(s == 0.0, 1.0, s)
    return 1.0 / jnp.sqrt(s)


_BR = 512
_BC = 1024
_NBR = _N // _BR
_NBC = _N // _BC


def _assemble_kernel(w1_ref, w2_ref,
                     rsi_ref, csi_ref, dgi_ref,
                     rsj_ref, csj_ref, dgj_ref,
                     a_ref):
    bi = pl.program_id(0)
    bj = pl.program_id(1)
    w2t = jnp.transpose(w2_ref[...])
    dni = _dn(rsi_ref[...], csi_ref[...], dgi_ref[...])
    dnj = _dn(rsj_ref[...].reshape(1, _BC),
              csj_ref[...].reshape(1, _BC),
              dgj_ref[...].reshape(1, _BC))
    offd = -_ALPHA * (dni * dnj) * (w1_ref[...].astype(jnp.float32) +
                                    w2t.astype(jnp.float32))
    irow = jax.lax.broadcasted_iota(jnp.int32, (_BR, _BC), 0) + bi * _BR
    jcol = jax.lax.broadcasted_iota(jnp.int32, (_BR, _BC), 1) + bj * _BC
    a_ref[...] = jnp.where(irow == jcol, 1.0, offd)


@jax.jit
def kernel(queries, keys):
    f32 = jnp.float32
    k2c = pl.pallas_call(
        _norms_kernel,
        out_shape=jax.ShapeDtypeStruct((_N, 1), f32),
    )(keys)
    k2row = k2c.reshape(1, _N)

    col = pl.BlockSpec((_BLK, 1), lambda bi: (bi, 0))
    stats = pl.pallas_call(
        _stats_kernel,
        grid=(_NB,),
        in_specs=[
            pl.BlockSpec((_BLK, _D), lambda bi: (bi, 0)),
            pl.BlockSpec((_N, _D), lambda bi: (0, 0)),
            pl.BlockSpec((1, _N), lambda bi: (0, 0)),
        ],
        out_specs=[col, col,
                   pl.BlockSpec((1, _N), lambda bi: (0, 0)),
                   pl.BlockSpec((_BLK, _N), lambda bi: (bi, 0))],
        out_shape=[jax.ShapeDtypeStruct((_N, 1), f32),
                   jax.ShapeDtypeStruct((_N, 1), f32),
                   jax.ShapeDtypeStruct((1, _N), f32),
                   jax.ShapeDtypeStruct((_N, _N), jnp.bfloat16)],
        scratch_shapes=[pltpu.VMEM((_BLK, _N), f32),
                        pltpu.VMEM((_BLK, _N), f32)],
        compiler_params=pltpu.CompilerParams(
            dimension_semantics=("arbitrary",)),
    )(queries, keys, k2row)
    rsc, dgc, csrow, w0 = stats
    csc = csrow.reshape(_N, 1)
    rsr = rsc.reshape(_NBC, 1, _BC)
    csr = csc.reshape(_NBC, 1, _BC)
    dgr = dgc.reshape(_NBC, 1, _BC)

    col_i = pl.BlockSpec((_BR, 1), lambda bi, bj: (bi, 0))
    row_j = pl.BlockSpec((1, 1, _BC), lambda bi, bj: (bj, 0, 0))

    a = pl.pallas_call(
        _assemble_kernel,
        grid=(_NBR, _NBC),
        in_specs=[pl.BlockSpec((_BR, _BC), lambda bi, bj: (bi, bj)),
                  pl.BlockSpec((_BC, _BR), lambda bi, bj: (bj, bi)),
                  col_i, col_i, col_i,
                  row_j, row_j, row_j],
        out_specs=pl.BlockSpec((_BR, _BC), lambda bi, bj: (bi, bj)),
        out_shape=jax.ShapeDtypeStruct((_N, _N), f32),
        compiler_params=pltpu.CompilerParams(
            dimension_semantics=("arbitrary", "arbitrary")),
    )(w0, w0,
      rsc, csc, dgc,
      rsr, csr, dgr)
    return a
